# baseline (speedup 1.0000x reference)
import functools

import jax
import jax.numpy as jnp
from jax import lax
from jax.experimental import pallas as pl
from jax.experimental.pallas import tpu as pltpu
from jax.experimental.pallas import tpu_sc as plsc

ROWS = 16384
COLS = 201

_INFO = plsc.get_sparse_core_info()
NC, NS, L = _INFO.num_cores, _INFO.num_subcores, _INFO.num_lanes
NW = NC * NS
RPW = ROWS // NW
CHUNK = 64
NCHUNK = RPW // CHUNK
GROUPS = CHUNK // L
LN2 = 0.6931471805599453
UNROLL = 8
_DMA_ONLY = True


def _ln(x):
    bits = plsc.bitcast(x, jnp.int32)
    e = (bits >> 23) - 127
    m = plsc.bitcast((bits & 0x007FFFFF) | 0x3F800000, jnp.float32)
    big = m > 1.4142135
    m = jnp.where(big, 0.5 * m, m)
    e = jnp.where(big, e + 1, e)
    z = (m - 1.0) / (m + 1.0)
    z2 = z * z
    p = z * (2.0 + z2 * (2.0 / 3.0 + z2 * (2.0 / 5.0 + z2 * (2.0 / 7.0 + z2 * (2.0 / 9.0)))))
    return e.astype(jnp.float32) * LN2 + p


def _group(buf, outv, out_off, g):
    rowv = lax.iota(jnp.int32, L) + g * L
    zero = jnp.zeros((L,), jnp.float32)
    y0 = plsc.load_gather(buf, [rowv, jnp.zeros((L,), jnp.int32)])
    pos = jnp.exp(y0)

    def body4(i, accs):
        cb = jnp.full((L,), 1 + UNROLL * i, jnp.int32)
        a0, a1, a2, a3 = accs
        for u in range(UNROLL):
            v = plsc.load_gather(buf, [rowv, cb + u])
            t = jnp.where(v > 0.0, jnp.exp(v), zero)
            if u % 4 == 0:
                a0 = a0 + t
            elif u % 4 == 1:
                a1 = a1 + t
            elif u % 4 == 2:
                a2 = a2 + t
            else:
                a3 = a3 + t
        return (a0, a1, a2, a3)

    if _DMA_ONLY:
        outv[pl.ds(out_off + g * L, L)] = pos
        return
    a0, a1, a2, a3 = lax.fori_loop(
        0, (COLS - 1) // UNROLL, body4, (zero, zero, zero, zero))
    acc = (a0 + a1) + (a2 + a3)
    outv[pl.ds(out_off + g * L, L)] = _ln((pos + acc) / pos)


NBUF = 4


def _body(y_hbm, out_hbm, buf0, buf1, buf2, buf3, outv, sem0, sem1, sem2, sem3):
    wid = lax.axis_index("s") * NC + lax.axis_index("c")
    base = wid * RPW
    bufs = (buf0, buf1, buf2, buf3)
    sems = (sem0, sem1, sem2, sem3)
    copies = []
    for c in range(NCHUNK):
        copies.append(pltpu.make_async_copy(
            y_hbm.at[pl.ds(base + c * CHUNK, CHUNK), :],
            bufs[c % NBUF], sems[c % NBUF]))
    for c in range(NBUF):
        copies[c].start()
    for c in range(NCHUNK):
        copies[c].wait()
        if c + NBUF < NCHUNK:
            copies[c + NBUF].start()
        for g in range(GROUPS):
            _group(bufs[c % NBUF], outv, c * CHUNK, g)
    pltpu.sync_copy(outv, out_hbm.at[pl.ds(base, RPW)])


@jax.jit
def _run(y):
    mesh = plsc.VectorSubcoreMesh(core_axis_name="c", subcore_axis_name="s")
    return pl.kernel(
        _body,
        out_type=jax.ShapeDtypeStruct((ROWS,), jnp.float32),
        mesh=mesh,
        compiler_params=pltpu.CompilerParams(needs_layout_passes=False),
        scratch_types=[
            pltpu.VMEM((CHUNK, COLS), jnp.float32),
            pltpu.VMEM((CHUNK, COLS), jnp.float32),
            pltpu.VMEM((CHUNK, COLS), jnp.float32),
            pltpu.VMEM((CHUNK, COLS), jnp.float32),
            pltpu.VMEM((RPW,), jnp.float32),
            pltpu.SemaphoreType.DMA,
            pltpu.SemaphoreType.DMA,
            pltpu.SemaphoreType.DMA,
            pltpu.SemaphoreType.DMA,
        ],
    )(y)


def kernel(y_pred, mask_zeros, temperature_):
    del mask_zeros, temperature_
    loss = _run(y_pred)
    return (loss, 0.0)


# speedup vs baseline: 1.3009x; 1.3009x over previous
import jax
import jax.numpy as jnp
from jax import lax
from jax.experimental import pallas as pl
from jax.experimental.pallas import tpu as pltpu

ROWS = 16384
COLS = 201
BLK = 2048


def _body(inv_t_ref, y_ref, o_ref):
    inv_t = inv_t_ref[0]
    y = y_ref[...] * inv_t
    e = jnp.exp(y)
    col = lax.broadcasted_iota(jnp.int32, (BLK, COLS), 1)
    keep = (col == 0) | (y > 0.0)
    c = jnp.where(keep, e, 0.0)
    ones = jnp.ones((COLS, 1), jnp.float32)
    s = lax.dot_general(c, ones, (((1,), (0,)), ((), ())),
                        preferred_element_type=jnp.float32)
    o_ref[...] = jnp.log(s[:, 0]) - y[:, 0]


def kernel(y_pred, mask_zeros, temperature_):
    del mask_zeros
    inv_t = (1.0 / temperature_).astype(jnp.float32)
    grid = (ROWS // BLK,)
    out = pl.pallas_call(
        _body,
        grid=grid,
        in_specs=[
            pl.BlockSpec(memory_space=pltpu.SMEM),
            pl.BlockSpec((BLK, COLS), lambda i: (i, 0)),
        ],
        out_specs=pl.BlockSpec((BLK,), lambda i: (i,)),
        out_shape=jax.ShapeDtypeStruct((ROWS,), jnp.float32),
    )(inv_t, y_pred)
    return (out, 0.0)


# speedup vs baseline: 1.4171x; 1.0893x over previous
import jax
import jax.numpy as jnp
from jax import lax
from jax.experimental import pallas as pl
from jax.experimental.pallas import tpu as pltpu

ROWS = 16384
COLS = 201
BLK = 2048

_DOT_T = (((1,), (1,)), ((), ()))


def _body(inv_t_ref, y_ref, o_ref):
    inv_t = inv_t_ref[0]
    y = y_ref[...] * inv_t
    e = jnp.exp(y)
    col = lax.broadcasted_iota(jnp.int32, (BLK, COLS), 1)
    keep = (col == 0) | (y > 0.0)
    c = jnp.where(keep, e, 0.0)
    ones = jnp.ones((1, COLS), jnp.float32)
    e1 = (lax.broadcasted_iota(jnp.int32, (1, COLS), 1) == 0).astype(jnp.float32)
    s = lax.dot_general(ones, c, _DOT_T, preferred_element_type=jnp.float32)
    y0 = lax.dot_general(e1, y, _DOT_T, preferred_element_type=jnp.float32)
    o_ref[...] = (jnp.log(s) - y0)[0]


def kernel(y_pred, mask_zeros, temperature_):
    del mask_zeros
    inv_t = (1.0 / temperature_).astype(jnp.float32)
    grid = (ROWS // BLK,)
    out = pl.pallas_call(
        _body,
        grid=grid,
        in_specs=[
            pl.BlockSpec(memory_space=pltpu.SMEM),
            pl.BlockSpec((BLK, COLS), lambda i: (i, 0)),
        ],
        out_specs=pl.BlockSpec((BLK,), lambda i: (i,)),
        out_shape=jax.ShapeDtypeStruct((ROWS,), jnp.float32),
    )(inv_t, y_pred)
    return (out, 0.0)
